# Initial kernel scaffold; baseline (speedup 1.0000x reference)
import jax
import jax.numpy as jnp
from jax.experimental import pallas as pl


def kernel(input_, tables):
    raise NotImplementedError("write your pallas kernel here")


# speedup vs baseline: 1.2090x; 1.2090x over previous
import functools

import jax
import jax.numpy as jnp
from jax import lax
from jax.experimental import pallas as pl
from jax.experimental.pallas import tpu as pltpu
from jax.experimental.pallas import tpu_sc as plsc

NUM_FIELDS = 26
VOCAB = 100000
EMBED_DIM = 32
BATCH = 16384

_INFO = plsc.get_sparse_core_info()
_NC, _NS, _L = _INFO.num_cores, _INFO.num_subcores, _INFO.num_lanes
_NW = _NC * _NS
_N = BATCH * NUM_FIELDS
_N_PER = _N // _NW
_CHUNK = 128
_NCHUNK = _N_PER // _CHUNK
_NBUF = 4


def _sc_body(table_hbm, idx_hbm, out_hbm, idx_v, gidx_v, rows_v, gsem, wsem):
    wid = lax.axis_index("s") * _NC + lax.axis_index("c")
    base = wid * _N_PER

    pltpu.sync_copy(idx_hbm.at[pl.ds(base, _N_PER)], idx_v)

    lane = lax.broadcasted_iota(jnp.int32, (_L,), 0)

    def compute(i, _):
        p = i * _L
        v = idx_v[pl.ds(p, _L)]
        f = lax.rem(p + lane, NUM_FIELDS)
        gidx_v[pl.ds(p, _L)] = v + f * VOCAB
        return 0

    lax.fori_loop(0, _N_PER // _L, compute, 0)

    def start_gather(c, b):
        pltpu.async_copy(table_hbm.at[gidx_v.at[pl.ds(c * _CHUNK, _CHUNK)]],
                         rows_v.at[b], gsem.at[b])

    def wait_gather(c, b):
        pltpu.make_async_copy(table_hbm.at[gidx_v.at[pl.ds(c * _CHUNK, _CHUNK)]],
                              rows_v.at[b], gsem.at[b]).wait()

    def start_wb(c, b):
        pltpu.make_async_copy(rows_v.at[b],
                              out_hbm.at[pl.ds(base + c * _CHUNK, _CHUNK)],
                              wsem.at[b]).start()

    def wait_wb(c, b):
        pltpu.make_async_copy(rows_v.at[b],
                              out_hbm.at[pl.ds(base + c * _CHUNK, _CHUNK)],
                              wsem.at[b]).wait()

    for c in range(_NBUF):
        start_gather(c, c)

    def body(c, _):
        @pl.when(c >= 1)
        def _():
            pb = (c - 1) % _NBUF
            wait_wb(c - 1, pb)

            @pl.when(c - 1 + _NBUF < _NCHUNK)
            def _():
                start_gather(c - 1 + _NBUF, pb)

        b = c % _NBUF
        wait_gather(c, b)
        start_wb(c, b)
        return 0

    lax.fori_loop(0, _NCHUNK, body, 0)
    wait_wb(_NCHUNK - 1, (_NCHUNK - 1) % _NBUF)


@jax.jit
def _run(table_flat, idx_flat):
    mesh = plsc.VectorSubcoreMesh(core_axis_name="c", subcore_axis_name="s")
    kern = functools.partial(
        pl.kernel,
        mesh=mesh,
        compiler_params=pltpu.CompilerParams(use_tc_tiling_on_sc=False),
        out_type=jax.ShapeDtypeStruct((_N, EMBED_DIM), jnp.float32),
        scratch_types=[
            pltpu.VMEM((_N_PER,), jnp.int32),
            pltpu.VMEM((_N_PER,), jnp.int32),
            pltpu.VMEM((_NBUF, _CHUNK, EMBED_DIM), jnp.float32),
            pltpu.SemaphoreType.DMA((_NBUF,)),
            pltpu.SemaphoreType.DMA((_NBUF,)),
        ],
    )(_sc_body)
    return kern(table_flat, idx_flat)


def kernel(input_, tables):
    table_flat = tables.reshape(NUM_FIELDS * VOCAB, EMBED_DIM)
    idx_flat = input_.reshape(_N)
    out = _run(table_flat, idx_flat)
    return out.reshape(BATCH, NUM_FIELDS * EMBED_DIM)
